# baseline (device time: 37822 ns/iter reference)
import jax
import jax.numpy as jnp
from jax import lax
from jax.experimental import pallas as pl
from jax.experimental.pallas import tpu as pltpu

MQ = 256
DC = 256
NC = 4


def kernel(dy, W):
    m, f = dy.shape
    d = W.shape[0]

    def body(
        dy_hbm,
        w_hbm,
        out_hbm,
        w_f32,
        out_vmem,
        w_bf,
        dy_f32,
        dy_bf,
        send_bf,
        recv_bf,
        recv_loc,
        w_sems,
        dy_sems,
        out_sems,
        x_send_sems,
        x_recv_sems,
        fwd_send_sems,
        fwd_recv_sems,
        loc_send_sems,
        loc_recv_sems,
        pair_bar,
    ):
        my_x = lax.axis_index("x")
        my_y = lax.axis_index("y")
        my_z = lax.axis_index("z")
        py = lax.rem(my_y, 2)
        pz = lax.rem(my_z, 2)
        x_peer = (1 - my_x, my_y, my_z)
        y_partner = (my_x, my_y + 1 - 2 * py, my_z)
        z_partner = (my_x, my_y, my_z + 1 - 2 * pz)

        jA = 2 * py + pz
        jD = 3 - jA
        order = (jA, jD, 2 * (1 - py) + pz, 2 * py + (1 - pz))

        def q(j):
            return pl.ds(j * MQ, MQ)

        def col(c):
            return pl.ds(c * DC, DC)

        for k, nbr in enumerate((x_peer, y_partner, z_partner)):
            pl.semaphore_signal(
                pair_bar.at[k], inc=1, device_id=nbr,
                device_id_type=pl.DeviceIdType.MESH,
            )

        w_cps = [
            pltpu.make_async_copy(
                w_hbm.at[col(c), :], w_f32.at[col(c), :], w_sems.at[c]
            )
            for c in range(NC)
        ]
        dy_cps = [
            pltpu.make_async_copy(
                dy_hbm.at[q(j), :], dy_f32.at[q(j), :], dy_sems.at[t]
            )
            for t, j in enumerate((jA, jD))
        ]
        w_cps[0].start()
        dy_cps[0].start()
        for c in range(1, NC):
            w_cps[c].start()
        dy_cps[1].start()

        def x_piece(t, j, c):
            return pltpu.make_async_remote_copy(
                src_ref=send_bf.at[q(j), col(c)],
                dst_ref=recv_bf.at[q(j), col(c)],
                send_sem=x_send_sems.at[t * NC + c],
                recv_sem=x_recv_sems.at[t * NC + c],
                device_id=x_peer,
                device_id_type=pl.DeviceIdType.MESH,
            )

        def fwd_piece(i, partner, c):
            return pltpu.make_async_remote_copy(
                src_ref=recv_bf.at[q(jA), col(c)],
                dst_ref=recv_bf.at[q(jA), col(c)],
                send_sem=fwd_send_sems.at[i * NC + c],
                recv_sem=fwd_recv_sems.at[i * NC + c],
                device_id=partner,
                device_id_type=pl.DeviceIdType.MESH,
            )

        def loc_piece(i, partner, c):
            return pltpu.make_async_remote_copy(
                src_ref=send_bf.at[q(jA), col(c)],
                dst_ref=recv_loc.at[q(jA), col(c)],
                send_sem=loc_send_sems.at[i * NC + c],
                recv_sem=loc_recv_sems.at[i * NC + c],
                device_id=partner,
                device_id_type=pl.DeviceIdType.MESH,
            )

        for t, j in enumerate((jA, jD)):
            dy_cps[t].wait()
            dy_bf[...] = dy_f32[q(j), :].astype(jnp.bfloat16)
            for c in range(NC):
                if t == 0:
                    w_cps[c].wait()
                    w_bf[col(c), :] = w_f32[col(c), :].astype(jnp.bfloat16)
                piece = lax.dot_general(
                    dy_bf[...],
                    w_bf[col(c), :],
                    dimension_numbers=(((1,), (1,)), ((), ())),
                    preferred_element_type=jnp.float32,
                )
                out_vmem[q(j), col(c)] = piece
                send_bf[q(j), col(c)] = piece.astype(jnp.bfloat16)
                if t == 0 and c == 0:
                    pl.semaphore_wait(pair_bar.at[0], 1)
                x_piece(t, j, c).start()
                if t == 0:
                    if c == 0:
                        pl.semaphore_wait(pair_bar.at[1], 1)
                        pl.semaphore_wait(pair_bar.at[2], 1)
                    for i, partner in enumerate((y_partner, z_partner)):
                        loc_piece(i, partner, c).start()
                if t == 1:
                    x_piece(0, jA, c).wait_recv()
                    for i, partner in enumerate((y_partner, z_partner)):
                        fwd_piece(i, partner, c).start()

        def out_cp(k, j):
            return pltpu.make_async_copy(
                out_vmem.at[q(j), :], out_hbm.at[q(j), :], out_sems.at[k]
            )

        def finish_quarter(k, j):
            out_vmem[q(j), :] = out_vmem[q(j), :] + recv_bf[q(j), :].astype(
                jnp.float32
            )
            out_cp(k, j).start()

        finish_quarter(0, jA)
        for c in range(NC):
            x_piece(1, jD, c).wait_recv()
        finish_quarter(1, jD)
        for i, (jP, partner) in enumerate(
            ((order[2], y_partner), (order[3], z_partner))
        ):
            for c in range(NC):
                loc_piece(i, partner, c).wait_recv()
                fwd_piece(i, partner, c).wait_recv()
            out_vmem[q(jP), :] = recv_loc[q(jP), :].astype(
                jnp.float32
            ) + recv_bf[q(jP), :].astype(jnp.float32)
            out_cp(2 + i, jP).start()

        for k, j in enumerate((jA, jD, order[2], order[3])):
            out_cp(k, j).wait()
        for t, j in enumerate((jA, jD)):
            for c in range(NC):
                x_piece(t, j, c).wait_send()
        for i, partner in enumerate((y_partner, z_partner)):
            for c in range(NC):
                fwd_piece(i, partner, c).wait_send()
                loc_piece(i, partner, c).wait_send()

    return pl.pallas_call(
        body,
        out_shape=jax.ShapeDtypeStruct((m, d), jnp.float32),
        in_specs=[
            pl.BlockSpec(memory_space=pl.ANY),
            pl.BlockSpec(memory_space=pl.ANY),
        ],
        out_specs=pl.BlockSpec(memory_space=pl.ANY),
        scratch_shapes=[
            pltpu.VMEM((d, f), jnp.float32),
            pltpu.VMEM((m, d), jnp.float32),
            pltpu.VMEM((d, f), jnp.bfloat16),
            pltpu.VMEM((m, f), jnp.float32),
            pltpu.VMEM((MQ, f), jnp.bfloat16),
            pltpu.VMEM((m, d), jnp.bfloat16),
            pltpu.VMEM((m, d), jnp.bfloat16),
            pltpu.VMEM((m, d), jnp.bfloat16),
            pltpu.SemaphoreType.DMA((NC,)),
            pltpu.SemaphoreType.DMA((2,)),
            pltpu.SemaphoreType.DMA((4,)),
            pltpu.SemaphoreType.DMA((2 * NC,)),
            pltpu.SemaphoreType.DMA((2 * NC,)),
            pltpu.SemaphoreType.DMA((2 * NC,)),
            pltpu.SemaphoreType.DMA((2 * NC,)),
            pltpu.SemaphoreType.DMA((2 * NC,)),
            pltpu.SemaphoreType.DMA((2 * NC,)),
            pltpu.SemaphoreType.REGULAR((3,)),
        ],
        compiler_params=pltpu.CompilerParams(
            vmem_limit_bytes=100 * 1024 * 1024,
        ),
    )(dy, W)


# device time: 28602 ns/iter; 1.3224x vs baseline; 1.3224x over previous
import jax
import jax.numpy as jnp
from jax import lax
from jax.experimental import pallas as pl
from jax.experimental.pallas import tpu as pltpu

MQ = 256
DC = 256
NC = 4


def kernel(dy, W):
    m, f = dy.shape
    d = W.shape[0]

    def body(
        dy_hbm,
        w_hbm,
        out_hbm,
        w_f32,
        out_vmem,
        w_bf,
        dy_f32,
        dy_bf,
        send_bf,
        recv_bf,
        recv_loc,
        w_sems,
        dy_sems,
        out_sems,
        x_send_sems,
        x_recv_sems,
        fwd_send_sems,
        fwd_recv_sems,
        loc_send_sems,
        loc_recv_sems,
    ):
        my_x = lax.axis_index("x")
        my_y = lax.axis_index("y")
        my_z = lax.axis_index("z")
        py = lax.rem(my_y, 2)
        pz = lax.rem(my_z, 2)
        x_peer = (1 - my_x, my_y, my_z)
        y_partner = (my_x, my_y + 1 - 2 * py, my_z)
        z_partner = (my_x, my_y, my_z + 1 - 2 * pz)

        jA = 2 * py + pz
        jD = 3 - jA
        order = (jA, jD, 2 * (1 - py) + pz, 2 * py + (1 - pz))

        def q(j):
            return pl.ds(j * MQ, MQ)

        def col(c):
            return pl.ds(c * DC, DC)

        barrier = pltpu.get_barrier_semaphore()
        for nbr in (x_peer, y_partner, z_partner):
            pl.semaphore_signal(
                barrier, inc=1, device_id=nbr,
                device_id_type=pl.DeviceIdType.MESH,
            )

        w_cps = [
            pltpu.make_async_copy(
                w_hbm.at[col(c), :], w_f32.at[col(c), :], w_sems.at[c]
            )
            for c in range(NC)
        ]
        dy_cps = [
            pltpu.make_async_copy(
                dy_hbm.at[q(j), :], dy_f32.at[q(j), :], dy_sems.at[t]
            )
            for t, j in enumerate((jA, jD))
        ]
        w_cps[0].start()
        dy_cps[0].start()

        def x_piece(t, j, c):
            return pltpu.make_async_remote_copy(
                src_ref=send_bf.at[q(j), col(c)],
                dst_ref=recv_bf.at[q(j), col(c)],
                send_sem=x_send_sems.at[t * NC + c],
                recv_sem=x_recv_sems.at[t * NC + c],
                device_id=x_peer,
                device_id_type=pl.DeviceIdType.MESH,
            )

        def fwd_piece(i, partner, c):
            return pltpu.make_async_remote_copy(
                src_ref=recv_bf.at[q(jA), col(c)],
                dst_ref=recv_bf.at[q(jA), col(c)],
                send_sem=fwd_send_sems.at[i * NC + c],
                recv_sem=fwd_recv_sems.at[i * NC + c],
                device_id=partner,
                device_id_type=pl.DeviceIdType.MESH,
            )

        def loc_piece(i, partner, c):
            return pltpu.make_async_remote_copy(
                src_ref=send_bf.at[q(jA), col(c)],
                dst_ref=recv_loc.at[q(jA), col(c)],
                send_sem=loc_send_sems.at[i * NC + c],
                recv_sem=loc_recv_sems.at[i * NC + c],
                device_id=partner,
                device_id_type=pl.DeviceIdType.MESH,
            )

        for t, j in enumerate((jA, jD)):
            dy_cps[t].wait()
            dy_bf[...] = dy_f32[q(j), :].astype(jnp.bfloat16)
            for c in range(NC):
                if t == 0:
                    w_cps[c].wait()
                    if c == 0:
                        for c2 in range(1, NC):
                            w_cps[c2].start()
                        dy_cps[1].start()
                    w_bf[col(c), :] = w_f32[col(c), :].astype(jnp.bfloat16)
                piece = lax.dot_general(
                    dy_bf[...],
                    w_bf[col(c), :],
                    dimension_numbers=(((1,), (1,)), ((), ())),
                    preferred_element_type=jnp.float32,
                )
                out_vmem[q(j), col(c)] = piece
                send_bf[q(j), col(c)] = piece.astype(jnp.bfloat16)
                if t == 0 and c == 0:
                    pl.semaphore_wait(barrier, 3)
                x_piece(t, j, c).start()
                if t == 0:
                    for i, partner in enumerate((y_partner, z_partner)):
                        loc_piece(i, partner, c).start()
                if t == 1:
                    x_piece(0, jA, c).wait_recv()
                    for i, partner in enumerate((y_partner, z_partner)):
                        fwd_piece(i, partner, c).start()
                    out_vmem[q(jA), col(c)] = out_vmem[q(jA), col(c)] + (
                        recv_bf[q(jA), col(c)].astype(jnp.float32)
                    )

        def out_cp(k, j):
            return pltpu.make_async_copy(
                out_vmem.at[q(j), :], out_hbm.at[q(j), :], out_sems.at[k]
            )

        def finish_quarter(k, j):
            out_vmem[q(j), :] = out_vmem[q(j), :] + recv_bf[q(j), :].astype(
                jnp.float32
            )
            out_cp(k, j).start()

        out_cp(0, jA).start()
        for c in range(NC):
            x_piece(1, jD, c).wait_recv()
        finish_quarter(1, jD)
        for i, (jP, partner) in enumerate(
            ((order[2], y_partner), (order[3], z_partner))
        ):
            for c in range(NC):
                loc_piece(i, partner, c).wait_recv()
                fwd_piece(i, partner, c).wait_recv()
            out_vmem[q(jP), :] = recv_loc[q(jP), :].astype(
                jnp.float32
            ) + recv_bf[q(jP), :].astype(jnp.float32)
            out_cp(2 + i, jP).start()

        for k, j in enumerate((jA, jD, order[2], order[3])):
            out_cp(k, j).wait()
        for t, j in enumerate((jA, jD)):
            for c in range(NC):
                x_piece(t, j, c).wait_send()
        for i, partner in enumerate((y_partner, z_partner)):
            for c in range(NC):
                fwd_piece(i, partner, c).wait_send()
                loc_piece(i, partner, c).wait_send()

    return pl.pallas_call(
        body,
        out_shape=jax.ShapeDtypeStruct((m, d), jnp.float32),
        in_specs=[
            pl.BlockSpec(memory_space=pl.ANY),
            pl.BlockSpec(memory_space=pl.ANY),
        ],
        out_specs=pl.BlockSpec(memory_space=pl.ANY),
        scratch_shapes=[
            pltpu.VMEM((d, f), jnp.float32),
            pltpu.VMEM((m, d), jnp.float32),
            pltpu.VMEM((d, f), jnp.bfloat16),
            pltpu.VMEM((m, f), jnp.float32),
            pltpu.VMEM((MQ, f), jnp.bfloat16),
            pltpu.VMEM((m, d), jnp.bfloat16),
            pltpu.VMEM((m, d), jnp.bfloat16),
            pltpu.VMEM((m, d), jnp.bfloat16),
            pltpu.SemaphoreType.DMA((NC,)),
            pltpu.SemaphoreType.DMA((2,)),
            pltpu.SemaphoreType.DMA((4,)),
            pltpu.SemaphoreType.DMA((2 * NC,)),
            pltpu.SemaphoreType.DMA((2 * NC,)),
            pltpu.SemaphoreType.DMA((2 * NC,)),
            pltpu.SemaphoreType.DMA((2 * NC,)),
            pltpu.SemaphoreType.DMA((2 * NC,)),
            pltpu.SemaphoreType.DMA((2 * NC,)),
        ],
        compiler_params=pltpu.CompilerParams(
            collective_id=0,
            vmem_limit_bytes=100 * 1024 * 1024,
        ),
    )(dy, W)
